# Initial kernel scaffold; baseline (speedup 1.0000x reference)
import jax
import jax.numpy as jnp
from jax.experimental import pallas as pl


def kernel(fe, down0, down1, unpool_idx1, unpool_idx2, gemm1, gemm2, up0_W1, up0_b1, up0_W2, up0_b2, up1_W1, up1_b1, up1_W2, up1_b2):
    raise NotImplementedError("write your pallas kernel here")


# speedup vs baseline: 5.4864x; 5.4864x over previous
import functools

import jax
import jax.numpy as jnp
from jax import lax
from jax.experimental import pallas as pl
from jax.experimental.pallas import tpu as pltpu, tpu_sc as plsc

NW = 32
STEP_E = 32
CHUNK_I = STEP_E * 5


def _ceil_to(x, m):
    return (x + m - 1) // m * m


def _sc_gather(E_in, C, E_pad, E_up):
    S = E_pad // (NW * STEP_E)
    assert S * NW * STEP_E == E_pad
    n_chunks = E_pad // STEP_E

    mesh = plsc.VectorSubcoreMesh(core_axis_name="c", subcore_axis_name="s")
    scratch = [
        pltpu.VMEM((80,), jnp.int32),
        pltpu.VMEM((80,), jnp.int32),
        pltpu.VMEM((160, C), jnp.float32),
        pltpu.SemaphoreType.DMA,
        pltpu.SemaphoreType.DMA,
    ]
    if E_up:
        scratch = [pltpu.VMEM((E_up,), jnp.int32)] + scratch

    def body(table_hbm, idx_hbm, *rest):
        if E_up:
            up_hbm, out_hbm, up_v, ia, ib, rows, s0, s1 = rest
            pltpu.sync_copy(up_hbm, up_v)
        else:
            out_hbm, ia, ib, rows, s0, s1 = rest
        wid = lax.axis_index("s") * 2 + lax.axis_index("c")

        def step(s, _):
            c = wid * S + s
            e0 = pl.multiple_of(c * STEP_E, STEP_E)
            pltpu.sync_copy(idx_hbm.at[c, 0], ia)
            pltpu.sync_copy(idx_hbm.at[c, 1], ib)
            if E_up:
                for buf in (ia, ib):
                    for i in range(5):
                        sl = pl.ds(i * 16, 16)
                        buf[sl] = plsc.load_gather(up_v, [buf[sl]])
            d0 = pltpu.async_copy(table_hbm.at[ia], rows.at[pl.ds(0, 80)], s0)
            d1 = pltpu.async_copy(table_hbm.at[ib], rows.at[pl.ds(80, 80)], s1)
            d0.wait()
            d1.wait()
            for k in range(5):
                pltpu.sync_copy(rows.at[pl.ds(k * STEP_E, STEP_E)],
                                out_hbm.at[k, pl.ds(e0, STEP_E), :])
            return _

        lax.fori_loop(0, S, step, 0)

    return pl.kernel(
        body,
        out_type=jax.ShapeDtypeStruct((5, E_pad, C), jnp.float32),
        mesh=mesh,
        scratch_types=scratch,
        compiler_params=pltpu.CompilerParams(needs_layout_passes=False,
                                             use_tc_tiling_on_sc=False),
    )


def _tc_conv(E_pad, Cin, Cout, cat_C, relu, Eb=512):
    nblk = E_pad // Eb
    assert nblk * Eb == E_pad

    def body(*refs):
        if cat_C:
            g_ref, w_ref, b_ref, d_ref, o_ref = refs
        else:
            g_ref, w_ref, b_ref, o_ref = refs
        f0, f1, f2, f3, f4 = (g_ref[k] for k in range(5))
        acc = jnp.dot(f0, w_ref[0], preferred_element_type=jnp.float32)
        acc += jnp.dot(f1 + f3, w_ref[1], preferred_element_type=jnp.float32)
        acc += jnp.dot(f2 + f4, w_ref[2], preferred_element_type=jnp.float32)
        acc += jnp.dot(jnp.abs(f1 - f3), w_ref[3], preferred_element_type=jnp.float32)
        acc += jnp.dot(jnp.abs(f2 - f4), w_ref[4], preferred_element_type=jnp.float32)
        acc += b_ref[0]
        if relu:
            acc = jnp.maximum(acc, 0.0)
        if cat_C:
            o_ref[...] = jnp.concatenate([acc, d_ref[...]], axis=-1)
        else:
            o_ref[...] = acc

    in_specs = [
        pl.BlockSpec((5, Eb, Cin), lambda i: (0, i, 0)),
        pl.BlockSpec((5, Cin, Cout), lambda i: (0, 0, 0)),
        pl.BlockSpec((1, Cout), lambda i: (0, 0)),
    ]
    if cat_C:
        in_specs.append(pl.BlockSpec((Eb, cat_C), lambda i: (i, 0)))

    return pl.pallas_call(
        body,
        grid=(nblk,),
        in_specs=in_specs,
        out_specs=pl.BlockSpec((Eb, Cout + cat_C), lambda i: (i, 0)),
        out_shape=jax.ShapeDtypeStruct((E_pad, Cout + cat_C), jnp.float32),
    )


def _prep_idx(g, E_pad):
    E = g.shape[0]
    gp = jnp.zeros((E_pad, 5), jnp.int32).at[:E].set(g.astype(jnp.int32))
    return gp.reshape(E_pad // STEP_E, STEP_E, 5).transpose(0, 2, 1).reshape(
        E_pad // STEP_E, 2, 80)


def _pad_rows(xT, E_pad):
    E, C = xT.shape
    return jnp.zeros((E_pad, C), xT.dtype).at[:E].set(xT)


def _pack_w(W):
    return jnp.transpose(W, (2, 1, 0)).astype(jnp.float32)


def kernel(fe, down0, down1, unpool_idx1, unpool_idx2, gemm1, gemm2,
           up0_W1, up0_b1, up0_W2, up0_b2, up1_W1, up1_b1, up1_W2, up1_b2):
    E0, E1, E2 = fe.shape[2], down0.shape[2], down1.shape[2]
    C0, C1, C2 = fe.shape[1], down0.shape[1], down1.shape[1]
    E1p = _ceil_to(E1, NW * STEP_E)
    E2p = _ceil_to(E2, NW * STEP_E)

    feT = fe[0].T
    d0T = _pad_rows(down0[0].T, E1p)
    d1T = _pad_rows(down1[0].T, E2p)
    idx1 = _prep_idx(gemm1[0], E1p)
    idx2 = _prep_idx(gemm2[0], E2p)
    up1 = unpool_idx1[0].astype(jnp.int32)
    up2 = unpool_idx2[0].astype(jnp.int32)
    W10, W20 = _pack_w(up0_W1), _pack_w(up0_W2)
    W11, W21 = _pack_w(up1_W1), _pack_w(up1_W2)
    b10, b20 = up0_b1.reshape(1, -1), up0_b2.reshape(1, -1)
    b11, b21 = up1_b1.reshape(1, -1), up1_b2.reshape(1, -1)

    G = _sc_gather(E0, C0, E1p, E1)(feT, idx1, up1)
    x = _tc_conv(E1p, C0, C1, C1, False)(G, W10, b10, d0T)
    G = _sc_gather(E1p, C0, E1p, 0)(x, idx1)
    x = _tc_conv(E1p, C0, C1, 0, True)(G, W10, b10)
    G = _sc_gather(E1p, C1, E1p, 0)(x, idx1)
    x = _tc_conv(E1p, C1, C1, 0, True)(G, W20, b20)

    G = _sc_gather(E1p, C1, E2p, E2)(x, idx2, up2)
    x = _tc_conv(E2p, C1, C2, C2, False)(G, W11, b11, d1T)
    G = _sc_gather(E2p, C1, E2p, 0)(x, idx2)
    x = _tc_conv(E2p, C1, C2, 0, True)(G, W11, b11)
    G = _sc_gather(E2p, C2, E2p, 0)(x, idx2)
    x = _tc_conv(E2p, C2, C2, 0, True)(G, W21, b21)

    return x[:E2].T[None]
